# baseline (speedup 1.0000x reference)
import functools

import jax
import jax.numpy as jnp
import numpy as np
from jax import lax
from jax.experimental import pallas as pl
from jax.experimental.pallas import tpu as pltpu
from jax.experimental.pallas import tpu_sc as plsc

B = 16384
EMB = 256
HALF = EMB // 2
PK = EMB // 2
SRC = 256
OUT = 2 * SRC

NC = 2
NS = 16
NW = NC * NS
B_PER_W = B // NW
CH = 128
NCH = B_PER_W // CH

BM = 4096


def _sc_gather_body(table_hbm, idx_hbm, out_hbm, idx_v, fb0, fb1, ib0, ib1,
                    sem_g, sem_s0, sem_s1):
    wid = lax.axis_index("s") * NC + lax.axis_index("c")
    base = wid * B_PER_W
    pltpu.sync_copy(idx_hbm.at[pl.ds(base, B_PER_W)], idx_v)
    fbufs = (fb0, fb1)
    ibufs = (ib0, ib1)
    sems = (sem_s0, sem_s1)
    scat = [None, None]

    def _convert(fbuf, ibuf):
        @plsc.parallel_loop(0, CH, 1, unroll=4)
        def _row(r):
            for k in range(HALF // 16):
                a = fbuf[r, pl.ds(k * 16, 16)]
                b = fbuf[r, pl.ds(HALF + k * 16, 16)]
                packed = plsc.pack(a, b, format=plsc.PackFormat.INTERLEAVED)
                ibuf[r, pl.ds(k * 16, 16)] = plsc.bitcast(packed, jnp.int32)

    g = pltpu.async_copy(table_hbm.at[idx_v.at[pl.ds(0, CH)]], fb0, sem_g)
    for c in range(NCH):
        g.wait()
        if c + 1 < NCH:
            g = pltpu.async_copy(
                table_hbm.at[idx_v.at[pl.ds((c + 1) * CH, CH)]],
                fbufs[(c + 1) % 2], sem_g)
        if scat[c % 2] is not None:
            scat[c % 2].wait()
        _convert(fbufs[c % 2], ibufs[c % 2])
        scat[c % 2] = pltpu.async_copy(
            ibufs[c % 2], out_hbm.at[pl.ds(base + c * CH, CH)], sems[c % 2])
    scat[0].wait()
    scat[1].wait()


_sc_gather = pl.kernel(
    _sc_gather_body,
    out_type=jax.ShapeDtypeStruct((B, PK), jnp.int32),
    mesh=plsc.VectorSubcoreMesh(core_axis_name="c", subcore_axis_name="s"),
    compiler_params=pltpu.CompilerParams(needs_layout_passes=False),
    scratch_types=[
        pltpu.VMEM((B_PER_W,), jnp.int32),
        pltpu.VMEM((CH, EMB), jnp.float32),
        pltpu.VMEM((CH, EMB), jnp.float32),
        pltpu.VMEM((CH, PK), jnp.int32),
        pltpu.VMEM((CH, PK), jnp.int32),
        pltpu.SemaphoreType.DMA,
        pltpu.SemaphoreType.DMA,
        pltpu.SemaphoreType.DMA,
    ],
)

_DN = (((1,), (1,)), ((), ()))


def _mm_body(x_ref, w0_ref, w1_ref, o_ref):
    x = x_ref[...]
    lo = lax.bitcast_convert_type(lax.shift_left(x, 16), jnp.float32)
    hi = lax.bitcast_convert_type(
        lax.bitwise_and(x, jnp.int32(-65536)), jnp.float32)
    w0 = w0_ref[...]
    w1 = w1_ref[...]
    out0 = (lax.dot_general(lo, w0[:, :HALF], _DN,
                            preferred_element_type=jnp.float32)
            + lax.dot_general(hi, w0[:, HALF:], _DN,
                              preferred_element_type=jnp.float32))
    out1 = (lax.dot_general(lo, w1[:, :HALF], _DN,
                            preferred_element_type=jnp.float32)
            + lax.dot_general(hi, w1[:, HALF:], _DN,
                              preferred_element_type=jnp.float32))
    o_ref[...] = jnp.concatenate([out0, out1], axis=1)


_matmul = pl.pallas_call(
    _mm_body,
    grid=(B // BM,),
    in_specs=[
        pl.BlockSpec((BM, PK), lambda i: (i, 0)),
        pl.BlockSpec((SRC, EMB), lambda i: (0, 0)),
        pl.BlockSpec((SRC, EMB), lambda i: (0, 0)),
    ],
    out_specs=pl.BlockSpec((BM, OUT), lambda i: (i, 0)),
    out_shape=jax.ShapeDtypeStruct((B, OUT), jnp.float32),
)


@jax.jit
def _run(indexes, entity_table, W0, W1):
    packed = _sc_gather(entity_table, indexes)
    return _matmul(packed, W0, W1).reshape(B, 2, SRC)


def kernel(indexes, entity_table, W0, W1):
    return _run(indexes, entity_table, W0, W1)


# speedup vs baseline: 1.6332x; 1.6332x over previous
import functools

import jax
import jax.numpy as jnp
import numpy as np
from jax import lax
from jax.experimental import pallas as pl
from jax.experimental.pallas import tpu as pltpu
from jax.experimental.pallas import tpu_sc as plsc

B = 16384
EMB = 256
HALF = EMB // 2
PK = EMB // 2
SRC = 256
OUT = 2 * SRC

NC = 2
NS = 16
NW = NC * NS
B_PER_W = B // NW
CH = 128
NCH = B_PER_W // CH

BM = 4096


def _sc_gather_body(table_hbm, idx_hbm, out_hbm, idx_v, fb0, fb1, ib0, ib1,
                    sem_g, sem_s0, sem_s1):
    wid = lax.axis_index("s") * NC + lax.axis_index("c")
    base = wid * B_PER_W
    pltpu.sync_copy(idx_hbm.at[pl.ds(base, B_PER_W)], idx_v)
    fbufs = (fb0, fb1)
    ibufs = (ib0, ib1)
    sems = (sem_s0, sem_s1)
    scat = [None, None]

    def _convert(fbuf, ibuf):
        @plsc.parallel_loop(0, CH, 1, unroll=4)
        def _row(r):
            for k in range(HALF // 16):
                a = fbuf[r, pl.ds(k * 16, 16)]
                b = fbuf[r, pl.ds(HALF + k * 16, 16)]
                packed = plsc.pack(a, b, format=plsc.PackFormat.INTERLEAVED)
                ibuf[r, pl.ds(k * 16, 16)] = plsc.bitcast(packed, jnp.int32)

    g = pltpu.async_copy(table_hbm.at[idx_v.at[pl.ds(0, CH)]], fb0, sem_g)
    for c in range(NCH):
        g.wait()
        if c + 1 < NCH:
            g = pltpu.async_copy(
                table_hbm.at[idx_v.at[pl.ds((c + 1) * CH, CH)]],
                fbufs[(c + 1) % 2], sem_g)
        if scat[c % 2] is not None:
            scat[c % 2].wait()
        _convert(fbufs[c % 2], ibufs[c % 2])
        scat[c % 2] = pltpu.async_copy(
            ibufs[c % 2], out_hbm.at[pl.ds(base + c * CH, CH)], sems[c % 2])
    scat[0].wait()
    scat[1].wait()


_sc_gather = pl.kernel(
    _sc_gather_body,
    out_type=jax.ShapeDtypeStruct((B, PK), jnp.int32),
    mesh=plsc.VectorSubcoreMesh(core_axis_name="c", subcore_axis_name="s"),
    compiler_params=pltpu.CompilerParams(needs_layout_passes=False),
    scratch_types=[
        pltpu.VMEM((B_PER_W,), jnp.int32),
        pltpu.VMEM((CH, EMB), jnp.float32),
        pltpu.VMEM((CH, EMB), jnp.float32),
        pltpu.VMEM((CH, PK), jnp.int32),
        pltpu.VMEM((CH, PK), jnp.int32),
        pltpu.SemaphoreType.DMA,
        pltpu.SemaphoreType.DMA,
        pltpu.SemaphoreType.DMA,
    ],
)

_DN = (((1,), (1,)), ((), ()))


def _mm_body(x_ref, w0_ref, w1_ref, o_ref):
    x = x_ref[...]
    lo = lax.bitcast_convert_type(lax.shift_left(x, 16), jnp.float32)
    hi = lax.bitcast_convert_type(
        lax.bitwise_and(x, jnp.int32(-65536)), jnp.float32)
    w0 = w0_ref[...]
    w1 = w1_ref[...]
    out0 = (lax.dot_general(lo, w0[:, :HALF], _DN,
                            preferred_element_type=jnp.float32)
            + lax.dot_general(hi, w0[:, HALF:], _DN,
                              preferred_element_type=jnp.float32))
    out1 = (lax.dot_general(lo, w1[:, :HALF], _DN,
                            preferred_element_type=jnp.float32)
            + lax.dot_general(hi, w1[:, HALF:], _DN,
                              preferred_element_type=jnp.float32))
    o_ref[:, 0, :] = out0
    o_ref[:, 1, :] = out1


_matmul = pl.pallas_call(
    _mm_body,
    grid=(B // BM,),
    in_specs=[
        pl.BlockSpec((BM, PK), lambda i: (i, 0)),
        pl.BlockSpec((SRC, EMB), lambda i: (0, 0)),
        pl.BlockSpec((SRC, EMB), lambda i: (0, 0)),
    ],
    out_specs=pl.BlockSpec((BM, 2, SRC), lambda i: (i, 0, 0)),
    out_shape=jax.ShapeDtypeStruct((B, 2, SRC), jnp.float32),
)


@jax.jit
def _run(indexes, entity_table, W0, W1):
    packed = _sc_gather(entity_table, indexes)
    return _matmul(packed, W0, W1)


def kernel(indexes, entity_table, W0, W1):
    return _run(indexes, entity_table, W0, W1)


# speedup vs baseline: 1.6752x; 1.0258x over previous
import jax
import jax.numpy as jnp
from jax import lax
from jax.experimental import pallas as pl
from jax.experimental.pallas import tpu as pltpu
from jax.experimental.pallas import tpu_sc as plsc

B = 16384
EMB = 256
SRC = 256
NC = 2
NS = 16
NW = NC * NS
B_PER_W = B // NW
CH = 128
NCH = B_PER_W // CH
BM = 4096


def _sc_gather_body(table_hbm, idx_hbm, out_hbm, idx_v, fb0, fb1,
                    sem_g, sem_s0, sem_s1):
    wid = lax.axis_index("s") * NC + lax.axis_index("c")
    base = wid * B_PER_W
    pltpu.sync_copy(idx_hbm.at[pl.ds(base, B_PER_W)], idx_v)
    fbufs = (fb0, fb1)
    sems = (sem_s0, sem_s1)
    scat = [None, None]
    g = pltpu.async_copy(table_hbm.at[idx_v.at[pl.ds(0, CH)]], fb0, sem_g)
    for c in range(NCH):
        g.wait()
        if c + 1 < NCH:
            if scat[(c + 1) % 2] is not None:
                scat[(c + 1) % 2].wait()
            g = pltpu.async_copy(
                table_hbm.at[idx_v.at[pl.ds((c + 1) * CH, CH)]],
                fbufs[(c + 1) % 2], sem_g)
        scat[c % 2] = pltpu.async_copy(
            fbufs[c % 2], out_hbm.at[pl.ds(base + c * CH, CH)], sems[c % 2])
    scat[0].wait()
    scat[1].wait()


_sc_gather = pl.kernel(
    _sc_gather_body,
    out_type=jax.ShapeDtypeStruct((B, EMB), jnp.float32),
    mesh=plsc.VectorSubcoreMesh(core_axis_name="c", subcore_axis_name="s"),
    compiler_params=pltpu.CompilerParams(needs_layout_passes=False),
    scratch_types=[
        pltpu.VMEM((B_PER_W,), jnp.int32),
        pltpu.VMEM((CH, EMB), jnp.float32),
        pltpu.VMEM((CH, EMB), jnp.float32),
        pltpu.SemaphoreType.DMA,
        pltpu.SemaphoreType.DMA,
        pltpu.SemaphoreType.DMA,
    ],
)

_DN = (((1,), (1,)), ((), ()))


def _mm_body(x_ref, w0_ref, w1_ref, o_ref):
    x = x_ref[...]
    o_ref[:, 0, :] = lax.dot_general(x, w0_ref[...], _DN,
                                     preferred_element_type=jnp.float32)
    o_ref[:, 1, :] = lax.dot_general(x, w1_ref[...], _DN,
                                     preferred_element_type=jnp.float32)


_matmul = pl.pallas_call(
    _mm_body,
    grid=(B // BM,),
    in_specs=[
        pl.BlockSpec((BM, EMB), lambda i: (i, 0)),
        pl.BlockSpec((SRC, EMB), lambda i: (0, 0)),
        pl.BlockSpec((SRC, EMB), lambda i: (0, 0)),
    ],
    out_specs=pl.BlockSpec((BM, 2, SRC), lambda i: (i, 0, 0)),
    out_shape=jax.ShapeDtypeStruct((B, 2, SRC), jnp.float32),
)


@jax.jit
def _run(indexes, entity_table, W0, W1):
    emb = _sc_gather(entity_table, indexes)
    return _matmul(emb, W0, W1)


def kernel(indexes, entity_table, W0, W1):
    return _run(indexes, entity_table, W0, W1)
